# baseline (speedup 1.0000x reference)
import functools

import jax
import jax.numpy as jnp
from jax import lax
from jax.experimental import pallas as pl
from jax.experimental.pallas import tpu as pltpu
from jax.experimental.pallas import tpu_sc as plsc


_NC, _NS = 2, 16
_NW = _NC * _NS

_SA = 64
_SB = 56
_PAIR = _SA + _SB


@functools.partial(jax.jit, static_argnames=("rows", "cols", "d"))
def _sc_gather(table, pos, *, rows, cols, d):
    b = rows * cols
    b_per_w = b // _NW
    w_per_row = cols // b_per_w
    n_pairs = (b_per_w - _SA) // _PAIR
    assert n_pairs * _PAIR + _SA == b_per_w
    mesh = plsc.VectorSubcoreMesh(core_axis_name="c", subcore_axis_name="s")

    @functools.partial(
        pl.kernel,
        mesh=mesh,
        out_type=jax.ShapeDtypeStruct((b, d), jnp.float32),
        scratch_types=[
            pltpu.VMEM((b_per_w,), jnp.int32),
            pltpu.VMEM((_SA, d), jnp.float32),
            pltpu.VMEM((_SB, d), jnp.float32),
            pltpu.SemaphoreType.DMA,
            pltpu.SemaphoreType.DMA,
            pltpu.SemaphoreType.DMA,
            pltpu.SemaphoreType.DMA,
        ],
    )
    def k(table_hbm, pos_hbm, out_hbm, idx_v, b0, b1, g0, g1, s0, s1):
        wid = lax.axis_index("s") * _NC + lax.axis_index("c")
        base = pl.multiple_of(wid * b_per_w, 8)
        col = pl.multiple_of((wid % w_per_row) * b_per_w, 8)
        pltpu.sync_copy(
            pos_hbm.at[wid // w_per_row, pl.ds(col, b_per_w)], idx_v
        )

        bufs = (b0, b1)
        sizes = (_SA, _SB)
        gsems = (g0, g1)
        ssems = (s0, s1)

        def gather_start(slot, off):
            off = pl.multiple_of(off, 8)
            pltpu.async_copy(
                table_hbm.at[idx_v.at[pl.ds(off, sizes[slot])]],
                bufs[slot],
                gsems[slot],
            )

        def gather_wait(slot):
            pltpu.make_async_copy(
                table_hbm.at[pl.ds(0, sizes[slot])], bufs[slot], gsems[slot]
            ).wait()

        def store_start(slot, off):
            row = pl.multiple_of(base + off, 8)
            pltpu.async_copy(
                bufs[slot], out_hbm.at[pl.ds(row, sizes[slot])], ssems[slot]
            )

        def store_wait(slot):
            pltpu.make_async_copy(
                bufs[slot], out_hbm.at[pl.ds(base, sizes[slot])], ssems[slot]
            ).wait()

        gather_start(0, 0)
        gather_start(1, _SA)
        gather_wait(0)
        store_start(0, 0)
        gather_wait(1)
        store_start(1, _SA)

        def step(i, carry):
            off_pair = pl.multiple_of((i + 1) * _PAIR, 8)
            store_wait(0)
            gather_start(0, off_pair)
            store_wait(1)
            gather_start(1, off_pair + _SA)
            gather_wait(0)
            store_start(0, off_pair)
            gather_wait(1)
            store_start(1, off_pair + _SA)
            return carry

        lax.fori_loop(0, n_pairs - 1, step, 0)

        tail_off = n_pairs * _PAIR
        store_wait(0)
        gather_start(0, tail_off)
        gather_wait(0)
        store_start(0, tail_off)
        store_wait(1)
        store_wait(0)

    return k(table, pos)


def kernel(pos, pos_embedding):
    rows, cols = pos.shape
    d = pos_embedding.shape[1]
    out = _sc_gather(
        pos_embedding, pos.astype(jnp.int32), rows=rows, cols=cols, d=d
    )
    return out.reshape(rows, cols, d)


# speedup vs baseline: 1.0489x; 1.0489x over previous
import functools

import jax
import jax.numpy as jnp
from jax import lax
from jax.experimental import pallas as pl
from jax.experimental.pallas import tpu as pltpu
from jax.experimental.pallas import tpu_sc as plsc


_NC, _NS = 2, 16
_NW = _NC * _NS

_CHUNK = 32
_NBUF = 3


@functools.partial(jax.jit, static_argnames=("rows", "cols", "d"))
def _sc_gather(table, pos, *, rows, cols, d):
    b = rows * cols
    b_per_w = b // _NW
    w_per_row = cols // b_per_w
    nch = b_per_w // _CHUNK
    mesh = plsc.VectorSubcoreMesh(core_axis_name="c", subcore_axis_name="s")

    @functools.partial(
        pl.kernel,
        mesh=mesh,
        out_type=jax.ShapeDtypeStruct((b, d), jnp.float32),
        scratch_types=[
            pltpu.VMEM((b_per_w,), jnp.int32),
            pltpu.VMEM((_CHUNK, d), jnp.float32),
            pltpu.VMEM((_CHUNK, d), jnp.float32),
            pltpu.VMEM((_CHUNK, d), jnp.float32),
            pltpu.SemaphoreType.DMA,
            pltpu.SemaphoreType.DMA,
            pltpu.SemaphoreType.DMA,
            pltpu.SemaphoreType.DMA,
            pltpu.SemaphoreType.DMA,
            pltpu.SemaphoreType.DMA,
        ],
    )
    def k(table_hbm, pos_hbm, out_hbm, idx_v, b0, b1, b2, g0, g1, g2, s0, s1, s2):
        wid = lax.axis_index("s") * _NC + lax.axis_index("c")
        base = pl.multiple_of(wid * b_per_w, 8)
        col = pl.multiple_of((wid % w_per_row) * b_per_w, 8)
        pltpu.sync_copy(
            pos_hbm.at[wid // w_per_row, pl.ds(col, b_per_w)], idx_v
        )

        bufs = (b0, b1, b2)
        gsems = (g0, g1, g2)
        ssems = (s0, s1, s2)

        def gather_start(slot, ch):
            off = pl.multiple_of(ch * _CHUNK, 8)
            pltpu.async_copy(
                table_hbm.at[idx_v.at[pl.ds(off, _CHUNK)]],
                bufs[slot],
                gsems[slot],
            )

        def gather_wait(slot):
            pltpu.make_async_copy(
                table_hbm.at[pl.ds(0, _CHUNK)], bufs[slot], gsems[slot]
            ).wait()

        def store_start(slot, ch):
            row = pl.multiple_of(base + ch * _CHUNK, 8)
            pltpu.async_copy(
                bufs[slot], out_hbm.at[pl.ds(row, _CHUNK)], ssems[slot]
            )

        def store_wait(slot):
            pltpu.make_async_copy(
                bufs[slot], out_hbm.at[pl.ds(base, _CHUNK)], ssems[slot]
            ).wait()

        def pipe_step(t, slot):
            if not isinstance(t, int) or t + 1 < nch:
                nxt = (slot + 1) % _NBUF

                def refill():
                    store_wait(nxt)
                    gather_start(nxt, t + 1)

                if isinstance(t, int):
                    if t + 1 >= _NBUF:
                        refill()
                    else:
                        gather_start(nxt, t + 1)
                else:
                    refill()
            gather_wait(slot)
            store_start(slot, t)

        gather_start(0, 0)
        n_main = ((nch - _NBUF) // _NBUF) * _NBUF
        for t in range(_NBUF):
            pipe_step(t, t % _NBUF)

        def step(i, carry):
            for j in range(_NBUF):
                t = _NBUF + i * _NBUF + j
                pipe_step(t, (_NBUF + j) % _NBUF)
            return carry

        lax.fori_loop(0, n_main // _NBUF, step, 0)
        for t in range(_NBUF + n_main, nch):
            pipe_step(t, t % _NBUF)
        for slot in range(_NBUF):
            store_wait(slot)

    return k(table, pos)


def kernel(pos, pos_embedding):
    rows, cols = pos.shape
    d = pos_embedding.shape[1]
    out = _sc_gather(
        pos_embedding, pos.astype(jnp.int32), rows=rows, cols=cols, d=d
    )
    return out.reshape(rows, cols, d)


# speedup vs baseline: 1.0533x; 1.0043x over previous
import functools

import jax
import jax.numpy as jnp
from jax import lax
from jax.experimental import pallas as pl
from jax.experimental.pallas import tpu as pltpu
from jax.experimental.pallas import tpu_sc as plsc


_NC, _NS = 2, 16
_NW = _NC * _NS

_CHUNK = 32
_NBUF = 2


@functools.partial(jax.jit, static_argnames=("rows", "cols", "d"))
def _sc_gather(table, pos, *, rows, cols, d):
    b = rows * cols
    b_per_w = b // _NW
    w_per_row = cols // b_per_w
    nch = b_per_w // _CHUNK
    mesh = plsc.VectorSubcoreMesh(core_axis_name="c", subcore_axis_name="s")

    @functools.partial(
        pl.kernel,
        mesh=mesh,
        out_type=jax.ShapeDtypeStruct((b, d), jnp.float32),
        scratch_types=[
            pltpu.VMEM((b_per_w,), jnp.int32),
            pltpu.VMEM((_CHUNK, d), jnp.float32),
            pltpu.VMEM((_CHUNK, d), jnp.float32),
            pltpu.VMEM((_CHUNK, d), jnp.float32),
            pltpu.SemaphoreType.DMA,
            pltpu.SemaphoreType.DMA,
            pltpu.SemaphoreType.DMA,
            pltpu.SemaphoreType.DMA,
            pltpu.SemaphoreType.DMA,
            pltpu.SemaphoreType.DMA,
        ],
    )
    def k(table_hbm, pos_hbm, out_hbm, idx_v, b0, b1, b2, g0, g1, g2, s0, s1, s2):
        wid = lax.axis_index("s") * _NC + lax.axis_index("c")
        base = pl.multiple_of(wid * b_per_w, 8)
        col = pl.multiple_of((wid % w_per_row) * b_per_w, 8)
        pltpu.sync_copy(
            pos_hbm.at[wid // w_per_row, pl.ds(col, b_per_w)], idx_v
        )

        bufs = (b0, b1, b2)
        gsems = (g0, g1, g2)
        ssems = (s0, s1, s2)

        def gather_start(slot, ch):
            off = pl.multiple_of(ch * _CHUNK, 8)
            pltpu.async_copy(
                table_hbm.at[idx_v.at[pl.ds(off, _CHUNK)]],
                bufs[slot],
                gsems[slot],
            )

        def gather_wait(slot):
            pltpu.make_async_copy(
                table_hbm.at[pl.ds(0, _CHUNK)], bufs[slot], gsems[slot]
            ).wait()

        def store_start(slot, ch):
            row = pl.multiple_of(base + ch * _CHUNK, 8)
            pltpu.async_copy(
                bufs[slot], out_hbm.at[pl.ds(row, _CHUNK)], ssems[slot]
            )

        def store_wait(slot):
            pltpu.make_async_copy(
                bufs[slot], out_hbm.at[pl.ds(base, _CHUNK)], ssems[slot]
            ).wait()

        def pipe_step(t, slot):
            if not isinstance(t, int) or t + 1 < nch:
                nxt = (slot + 1) % _NBUF

                def refill():
                    store_wait(nxt)
                    gather_start(nxt, t + 1)

                if isinstance(t, int):
                    if t + 1 >= _NBUF:
                        refill()
                    else:
                        gather_start(nxt, t + 1)
                else:
                    refill()
            gather_wait(slot)
            store_start(slot, t)

        gather_start(0, 0)
        n_main = ((nch - _NBUF - 1) // _NBUF) * _NBUF
        for t in range(_NBUF):
            pipe_step(t, t % _NBUF)

        def step(i, carry):
            for j in range(_NBUF):
                t = _NBUF + i * _NBUF + j
                pipe_step(t, (_NBUF + j) % _NBUF)
            return carry

        lax.fori_loop(0, n_main // _NBUF, step, 0)
        for t in range(_NBUF + n_main, nch):
            pipe_step(t, t % _NBUF)
        for slot in range(_NBUF):
            store_wait(slot)

    return k(table, pos)


def kernel(pos, pos_embedding):
    rows, cols = pos.shape
    d = pos_embedding.shape[1]
    out = _sc_gather(
        pos_embedding, pos.astype(jnp.int32), rows=rows, cols=cols, d=d
    )
    return out.reshape(rows, cols, d)
